# baseline (speedup 1.0000x reference)
import jax
import jax.numpy as jnp
from jax.experimental import pallas as pl
from jax.experimental.pallas import tpu as pltpu

USER_N = 6000
ITEM_N = 4000
NTOT = USER_N + ITEM_N
LAT = 32
HYP = 128
BM = 200
RBLKS = NTOT // BM
UBLKS = USER_N // BM
NB = 4
VLIM = 100 * 1024 * 1024

_F32 = jnp.float32


def _dotT(a, b):
    return jax.lax.dot_general(a, b, (((0,), (0,)), ((), ())),
                               preferred_element_type=_F32)


def _issue(adj_ref, abuf, sem, b):
    slot = jax.lax.rem(b, NB)
    pltpu.make_async_copy(
        adj_ref.at[pl.ds(b * BM, BM), :], abuf.at[slot], sem.at[slot]
    ).start()


def _wait(adj_ref, abuf, sem, r):
    slot = jax.lax.rem(r, NB)
    pltpu.make_async_copy(
        adj_ref.at[pl.ds(r * BM, BM), :], abuf.at[slot], sem.at[slot]
    ).wait()
    return abuf[slot]


def _layer1_body(adj_ref, emb_ref, embblk_ref, uH_ref, iH_ref,
                 tem_ref, h_ref, e1_ref, Lu_ref, Li_ref,
                 abuf, sem, Ku_s, Ki_s, Pu_s, Pi_s):
    r = pl.program_id(0)

    @pl.when(r == 0)
    def _prologue():
        for j in range(NB - 1):
            _issue(adj_ref, abuf, sem, j)
        uE = emb_ref[:USER_N, :]
        iE = emb_ref[USER_N:, :]
        Gu = _dotT(uE, uE)
        Gi = _dotT(iE, iE)
        Ku_s[...] = jnp.dot(uH_ref[...], _dotT(uH_ref[...], Gu),
                            preferred_element_type=_F32)
        Ki_s[...] = jnp.dot(iH_ref[...], _dotT(iH_ref[...], Gi),
                            preferred_element_type=_F32)
        Pu_s[...] = jnp.zeros_like(Pu_s)
        Pi_s[...] = jnp.zeros_like(Pi_s)

    @pl.when(r + NB - 1 < RBLKS)
    def _prefetch():
        _issue(adj_ref, abuf, sem, r + NB - 1)

    a = _wait(adj_ref, abuf, sem, r)
    tem = jnp.dot(a, emb_ref[...], preferred_element_type=_F32)
    eblk = embblk_ref[...]
    K = jnp.where(r < UBLKS, Ku_s[...], Ki_s[...])
    h = jnp.dot(eblk, K, preferred_element_type=_F32)
    e1 = tem + h
    tem_ref[...] = tem
    h_ref[...] = h
    e1_ref[...] = e1
    contrib = _dotT(eblk, e1)

    @pl.when(r < UBLKS)
    def _accu():
        Pu_s[...] += contrib

    @pl.when(r >= UBLKS)
    def _acci():
        Pi_s[...] += contrib

    @pl.when(r == RBLKS - 1)
    def _fin():
        Lu_ref[...] = jnp.dot(uH_ref[...], _dotT(uH_ref[...], Pu_s[...]),
                              preferred_element_type=_F32)
        Li_ref[...] = jnp.dot(iH_ref[...], _dotT(iH_ref[...], Pi_s[...]),
                              preferred_element_type=_F32)


def _layer2_body(adj_ref, e1_ref, embblk_ref, e1blk_ref, Lu_ref, Li_ref,
                 tem2_ref, h2_ref, out_ref,
                 abuf, sem):
    r = pl.program_id(0)

    @pl.when(r == 0)
    def _prologue():
        for j in range(NB - 1):
            _issue(adj_ref, abuf, sem, j)

    @pl.when(r + NB - 1 < RBLKS)
    def _prefetch():
        _issue(adj_ref, abuf, sem, r + NB - 1)

    a = _wait(adj_ref, abuf, sem, r)
    tem2 = jnp.dot(a, e1_ref[...], preferred_element_type=_F32)
    L = jnp.where(r < UBLKS, Lu_ref[...], Li_ref[...])
    eblk = embblk_ref[...]
    h2 = jnp.dot(eblk, L, preferred_element_type=_F32)
    tem2_ref[...] = tem2
    h2_ref[...] = h2
    out_ref[...] = eblk + e1blk_ref[...] + tem2 + h2


def _any_spec():
    return pl.BlockSpec(memory_space=pl.ANY)


def _full_spec(shape):
    return pl.BlockSpec(shape, lambda r: tuple(0 for _ in shape))


def _blk_spec():
    return pl.BlockSpec((BM, LAT), lambda r: (r, 0))


_layer1 = pl.pallas_call(
    _layer1_body,
    grid=(RBLKS,),
    in_specs=[
        _any_spec(),
        _full_spec((NTOT, LAT)),
        _blk_spec(),
        _full_spec((LAT, HYP)),
        _full_spec((LAT, HYP)),
    ],
    out_specs=[
        _blk_spec(),
        _blk_spec(),
        _blk_spec(),
        _full_spec((LAT, LAT)),
        _full_spec((LAT, LAT)),
    ],
    out_shape=[
        jax.ShapeDtypeStruct((NTOT, LAT), _F32),
        jax.ShapeDtypeStruct((NTOT, LAT), _F32),
        jax.ShapeDtypeStruct((NTOT, LAT), _F32),
        jax.ShapeDtypeStruct((LAT, LAT), _F32),
        jax.ShapeDtypeStruct((LAT, LAT), _F32),
    ],
    scratch_shapes=[
        pltpu.VMEM((NB, BM, NTOT), _F32),
        pltpu.SemaphoreType.DMA((NB,)),
        pltpu.VMEM((LAT, LAT), _F32),
        pltpu.VMEM((LAT, LAT), _F32),
        pltpu.VMEM((LAT, LAT), _F32),
        pltpu.VMEM((LAT, LAT), _F32),
    ],
    compiler_params=pltpu.CompilerParams(
        dimension_semantics=("arbitrary",),
        vmem_limit_bytes=VLIM),
)

_layer2 = pl.pallas_call(
    _layer2_body,
    grid=(RBLKS,),
    in_specs=[
        _any_spec(),
        _full_spec((NTOT, LAT)),
        _blk_spec(),
        _blk_spec(),
        _full_spec((LAT, LAT)),
        _full_spec((LAT, LAT)),
    ],
    out_specs=[_blk_spec(), _blk_spec(), _blk_spec()],
    out_shape=[
        jax.ShapeDtypeStruct((NTOT, LAT), _F32),
        jax.ShapeDtypeStruct((NTOT, LAT), _F32),
        jax.ShapeDtypeStruct((NTOT, LAT), _F32),
    ],
    scratch_shapes=[
        pltpu.VMEM((NB, BM, NTOT), _F32),
        pltpu.SemaphoreType.DMA((NB,)),
    ],
    compiler_params=pltpu.CompilerParams(
        dimension_semantics=("arbitrary",),
        vmem_limit_bytes=VLIM),
)


def kernel(adj, keepRate, uEmbeds, iEmbeds, uHyper, iHyper):
    del keepRate
    emb = jnp.concatenate([uEmbeds, iEmbeds], axis=0)
    tem1, h1, e1, Lu, Li = _layer1(adj, emb, emb, uHyper, iHyper)
    tem2, h2, out = _layer2(adj, e1, emb, e1, Lu, Li)
    return (out, tem1, tem2, h1, h2)
